# baseline (speedup 1.0000x reference)
import jax
import jax.numpy as jnp
from jax import lax
from jax.experimental import pallas as pl
from jax.experimental.pallas import tpu as pltpu
from jax.experimental.pallas import tpu_sc as plsc

NUM_CLASSES = 1000
B, S = 4096, 20
NC, NS, L = 2, 16, 16
NW = NC * NS
BPW = B // NW
GPW = BPW // L
CH = 200
NCH = NUM_CLASSES // CH
NBLK = S * NCH
NBUF = 2


def _body(xt_hbm, zb_hbm, out_hbm, idx_v, buf0, buf1, sem0, sem1, sem2):
  wid = lax.axis_index("s") * NC + lax.axis_index("c")
  b0 = wid * BPW

  idx_cp = pltpu.async_copy(xt_hbm.at[:, pl.ds(b0, BPW)], idx_v, sem0)
  zb0_cp = pltpu.async_copy(zb_hbm, buf0, sem1)
  zb1_cp = pltpu.async_copy(zb_hbm, buf1, sem2)

  zeros = jnp.zeros((L,), jnp.int32)
  ones = jnp.ones((L,), jnp.int32)
  bufs = (buf0, buf1)
  sems = (sem0, sem1)

  iota = lax.iota(jnp.int32, L)

  def _scatter(buf, j, c0, val):
    for g in range(GPW):
      cvec = idx_v[j, pl.ds(g * L, L)]
      mask = (cvec >= c0) & (cvec < c0 + CH)
      plsc.store_scatter(buf, [cvec - c0, iota + g * L], val, mask=mask)

  def _dst(tt):
    j = tt // NCH
    c0 = (tt - j * NCH) * CH
    return j, c0, out_hbm.at[j, pl.ds(c0, CH), pl.ds(b0, BPW)]

  zb0_cp.wait()
  idx_cp.wait()
  _scatter(buf0, 0, 0, ones)
  pltpu.async_copy(buf0, _dst(0)[2], sem0)
  zb1_cp.wait()
  _scatter(buf1, 0, CH, ones)
  pltpu.async_copy(buf1, _dst(1)[2], sem1)

  def _step(i, _):
    for b in range(NBUF):
      tt = i * NBUF + b
      buf = bufs[b]
      sem = sems[b]
      j, c0, dst = _dst(tt)
      pltpu.make_async_copy(buf, dst, sem).wait()
      jp, c0p, _ = _dst(tt - NBUF)
      _scatter(buf, jp, c0p, zeros)
      _scatter(buf, j, c0, ones)
      pltpu.async_copy(buf, dst, sem)
    return 0

  lax.fori_loop(1, NBLK // NBUF, _step, 0)

  for b in range(NBUF):
    tt = NBLK - NBUF + b
    pltpu.make_async_copy(bufs[b], _dst(tt)[2], sems[b]).wait()


@jax.jit
def _one_hot_sc(xt):
  mesh = plsc.VectorSubcoreMesh(core_axis_name="c", subcore_axis_name="s")
  k = pl.kernel(
      _body,
      out_type=jax.ShapeDtypeStruct((S, NUM_CLASSES, B), jnp.int32),
      mesh=mesh,
      scratch_types=[
          pltpu.VMEM((S, BPW), jnp.int32),
          pltpu.VMEM((CH, BPW), jnp.int32),
          pltpu.VMEM((CH, BPW), jnp.int32),
          pltpu.SemaphoreType.DMA,
          pltpu.SemaphoreType.DMA,
          pltpu.SemaphoreType.DMA,
      ],
      compiler_params=pltpu.CompilerParams(
          needs_layout_passes=False,
          use_tc_tiling_on_sc=True,
          disable_bounds_checks=True,
          disable_semaphore_checks=True,
          skip_device_barrier=True,
      ),
  )
  zb = jnp.zeros((CH, BPW), jnp.int32)
  return k(xt, zb)


def kernel(x):
  xt = x.astype(jnp.int32).T
  out_t = _one_hot_sc(xt)
  return jnp.transpose(out_t, (2, 0, 1))


# speedup vs baseline: 1.1448x; 1.1448x over previous
import jax
import jax.numpy as jnp
from jax import lax
from jax.experimental import pallas as pl
from jax.experimental.pallas import tpu as pltpu
from jax.experimental.pallas import tpu_sc as plsc

NUM_CLASSES = 1000
B, S = 4096, 20
NC, NS, L = 2, 16, 16
NW = NC * NS
BPW = B // NW
GPW = BPW // L
CH = 200
NCH = NUM_CLASSES // CH
NBLK = S * NCH
NBUF = 2


def _body(xt_hbm, out_hbm, idx_v, buf0, buf1, sem0, sem1):
  wid = lax.axis_index("s") * NC + lax.axis_index("c")
  b0 = wid * BPW

  idx_cp = pltpu.async_copy(xt_hbm.at[:, pl.ds(b0, BPW)], idx_v, sem0)

  zeros = jnp.zeros((L,), jnp.int32)
  ones = jnp.ones((L,), jnp.int32)
  bufs = (buf0, buf1)
  sems = (sem0, sem1)

  def _zero_fill(buf):
    def _zero(c, _):
      for g in range(GPW):
        buf[c, pl.ds(g * L, L)] = zeros
        buf[c + CH // 2, pl.ds(g * L, L)] = zeros
      return 0

    lax.fori_loop(0, CH // 2, _zero, 0)

  iota = lax.iota(jnp.int32, L)

  def _scatter(buf, j, c0, val):
    for g in range(GPW):
      cvec = idx_v[j, pl.ds(g * L, L)]
      mask = (cvec >= c0) & (cvec < c0 + CH)
      plsc.store_scatter(buf, [cvec - c0, iota + g * L], val, mask=mask)

  def _dst(tt):
    j = tt // NCH
    c0 = (tt - j * NCH) * CH
    return j, c0, out_hbm.at[j, pl.ds(c0, CH), pl.ds(b0, BPW)]

  _zero_fill(buf0)
  idx_cp.wait()
  _scatter(buf0, 0, 0, ones)
  pltpu.async_copy(buf0, _dst(0)[2], sem0)
  _zero_fill(buf1)
  _scatter(buf1, 0, CH, ones)
  pltpu.async_copy(buf1, _dst(1)[2], sem1)

  def _step(i, _):
    for b in range(NBUF):
      tt = i * NBUF + b
      buf = bufs[b]
      sem = sems[b]
      j, c0, dst = _dst(tt)
      pltpu.make_async_copy(buf, dst, sem).wait()
      jp, c0p, _ = _dst(tt - NBUF)
      _scatter(buf, jp, c0p, zeros)
      _scatter(buf, j, c0, ones)
      pltpu.async_copy(buf, dst, sem)
    return 0

  lax.fori_loop(1, NBLK // NBUF, _step, 0)

  for b in range(NBUF):
    tt = NBLK - NBUF + b
    pltpu.make_async_copy(bufs[b], _dst(tt)[2], sems[b]).wait()


@jax.jit
def _one_hot_sc(xt):
  mesh = plsc.VectorSubcoreMesh(core_axis_name="c", subcore_axis_name="s")
  k = pl.kernel(
      _body,
      out_type=jax.ShapeDtypeStruct((S, NUM_CLASSES, B), jnp.int32),
      mesh=mesh,
      scratch_types=[
          pltpu.VMEM((S, BPW), jnp.int32),
          pltpu.VMEM((CH, BPW), jnp.int32),
          pltpu.VMEM((CH, BPW), jnp.int32),
          pltpu.SemaphoreType.DMA,
          pltpu.SemaphoreType.DMA,
      ],
      compiler_params=pltpu.CompilerParams(
          needs_layout_passes=False,
          use_tc_tiling_on_sc=True,
      ),
  )
  return k(xt)


def kernel(x):
  xt = x.astype(jnp.int32).T
  out_t = _one_hot_sc(xt)
  return jnp.transpose(out_t, (2, 0, 1))
